# baseline (speedup 1.0000x reference)
import functools

import jax
import jax.numpy as jnp
from jax import lax
from jax.experimental import pallas as pl
from jax.experimental.pallas import tpu as pltpu
from jax.experimental.pallas import tpu_sc as plsc

_NC = 2
_NS = 16
_NW = _NC * _NS
_L = 16


def _kld_terms(dxo, dyo, dlo, dso, ao, dxt, dyt, dlt, dst, at_):
    aa = ao * ao + 1.0
    bb = at_ * at_ + 1.0
    cc = ao * at_ + 1.0
    ss = ao - at_
    elt = jnp.exp(dlt + dlt)
    est = jnp.exp(dst + dst)
    ielo = jnp.exp(-(dlo + dlo))
    ieso = jnp.exp(-(dso + dso))
    t12 = elt * ielo + est * ieso
    t34 = elt * ieso + est * ielo
    tn = cc * cc * t12 + ss * ss * t34
    dx = dxo - dxt
    dy = dyo - dyt
    u = dx + ao * dy
    v = dy - ao * dx
    dn = u * u * ielo + v * v * ieso
    qh = 0.5 / (aa * bb)
    det = (dlo - dlt) + (dso - dst)
    return (tn + (4.0 * bb) * dn) * qh + det


@functools.lru_cache(maxsize=None)
def _build_sc(B, N, b_lo):
    nt = N // 128
    npt = nt // _NW
    nb = (B - b_lo) // 2
    mesh = plsc.VectorSubcoreMesh(core_axis_name="c", subcore_axis_name="s")

    @functools.partial(
        pl.kernel,
        out_type=jax.ShapeDtypeStruct((_NW * 2 * _L,), jnp.float32),
        mesh=mesh,
        compiler_params=pltpu.CompilerParams(needs_layout_passes=False),
        scratch_types=[
            pltpu.VMEM((5 * npt, 128), jnp.float32),
            pltpu.VMEM((5 * npt, 128), jnp.float32),
            pltpu.VMEM((5 * npt, 128), jnp.float32),
            pltpu.VMEM((5 * npt, 128), jnp.float32),
            pltpu.VMEM((npt, 128), jnp.int32),
            pltpu.VMEM((npt, 128), jnp.int32),
            pltpu.VMEM((2 * _L,), jnp.float32),
            pltpu.SemaphoreType.DMA,
            pltpu.SemaphoreType.DMA,
        ],
    )
    def sc_kern(e_hbm, t_hbm, l_hbm, part_hbm,
                ebuf0, ebuf1, tbuf0, tbuf1, lbuf0, lbuf1,
                pbuf, sem0, sem1):
        wid = lax.axis_index("s") * _NC + lax.axis_index("c")
        tc0 = wid * npt

        def issue(b, eb, tb, lb, sem):
            tr = b >> 3
            sl = b & 7
            pltpu.async_copy(
                e_hbm.at[:, tr, pl.ds(tc0, npt), sl, :], eb.reshape(5, npt, 128), sem)
            pltpu.async_copy(
                t_hbm.at[:, tr, pl.ds(tc0, npt), sl, :], tb.reshape(5, npt, 128), sem)
            pltpu.async_copy(l_hbm.at[tr, pl.ds(tc0, npt), sl, :], lb, sem)

        def drain(eb, tb, lb, sem):
            pltpu.make_async_copy(
                e_hbm.at[:, 0, pl.ds(0, npt), 0, :], eb.reshape(5, npt, 128), sem).wait()
            pltpu.make_async_copy(
                t_hbm.at[:, 0, pl.ds(0, npt), 0, :], tb.reshape(5, npt, 128), sem).wait()
            pltpu.make_async_copy(l_hbm.at[0, pl.ds(0, npt), 0, :], lb, sem).wait()

        def compute(eb, tb, lb, acc):
            def tile_body(k, acc2):
                ak, ac = acc2
                i = k >> 1
                jb = (k & 1) * (4 * _L)
                for j4 in range(4):
                    j = jb + j4 * _L
                    kld = _kld_terms(
                        eb[i, pl.ds(j, _L)],
                        eb[i + npt, pl.ds(j, _L)],
                        eb[i + 2 * npt, pl.ds(j, _L)],
                        eb[i + 3 * npt, pl.ds(j, _L)],
                        eb[i + 4 * npt, pl.ds(j, _L)],
                        tb[i, pl.ds(j, _L)],
                        tb[i + npt, pl.ds(j, _L)],
                        tb[i + 2 * npt, pl.ds(j, _L)],
                        tb[i + 3 * npt, pl.ds(j, _L)],
                        tb[i + 4 * npt, pl.ds(j, _L)],
                    )
                    lf = lb[i, pl.ds(j, _L)].astype(jnp.float32)
                    ak = ak + kld * lf
                    ac = ac + lf
                return (ak, ac)

            return lax.fori_loop(0, npt * 2, tile_body, acc)

        zero = jnp.zeros((_L,), jnp.float32)
        issue(b_lo, ebuf0, tbuf0, lbuf0, sem0)

        def g_body(g, acc):
            b0 = b_lo + g * 2
            issue(b0 + 1, ebuf1, tbuf1, lbuf1, sem1)
            drain(ebuf0, tbuf0, lbuf0, sem0)
            acc = compute(ebuf0, tbuf0, lbuf0, acc)

            @pl.when(b0 + 2 < B)
            def _():
                issue(b0 + 2, ebuf0, tbuf0, lbuf0, sem0)

            drain(ebuf1, tbuf1, lbuf1, sem1)
            return compute(ebuf1, tbuf1, lbuf1, acc)

        acc_k, acc_c = lax.fori_loop(0, nb, g_body, (zero, zero))
        pbuf[pl.ds(0, _L)] = acc_k
        pbuf[pl.ds(_L, _L)] = acc_c
        pltpu.sync_copy(pbuf, part_hbm.at[pl.ds(wid * 2 * _L, 2 * _L)])

    return sc_kern


def _tc_body(e_ref, t_ref, l_ref, o_ref):
    @pl.when((pl.program_id(0) == 0) & (pl.program_id(1) == 0))
    def _():
        o_ref[0, 0] = 0.0
        o_ref[0, 1] = 0.0

    kld = _kld_terms(
        e_ref[0, 0], e_ref[1, 0], e_ref[2, 0], e_ref[3, 0], e_ref[4, 0],
        t_ref[0, 0], t_ref[1, 0], t_ref[2, 0], t_ref[3, 0], t_ref[4, 0],
    )
    lf = l_ref[0].astype(jnp.float32)
    o_ref[0, 0] += jnp.sum(kld * lf)
    o_ref[0, 1] += jnp.sum(lf)


def _finish_body(p_ref, tc_ref, o_ref):
    x = p_ref[...]
    lane = lax.broadcasted_iota(jnp.int32, x.shape, 1)
    is_k = (lane % (2 * _L)) < _L
    sk = jnp.sum(jnp.where(is_k, x, 0.0)) + tc_ref[0, 0]
    sc = jnp.sum(jnp.where(is_k, 0.0, x)) + tc_ref[0, 1]
    o_ref[0, 0] = sk / sc - 1.0


def kernel(out_ellipse, labels, ellipse_targets, anchors):
    B, N, F = out_ellipse.shape
    nt = N // 128
    nb = B // 8
    e5 = out_ellipse.transpose(2, 0, 1).reshape(F, nb, 8, nt, 128).transpose(0, 1, 3, 2, 4)
    t5 = ellipse_targets.transpose(2, 0, 1).reshape(F, nb, 8, nt, 128).transpose(0, 1, 3, 2, 4)
    l4 = labels.reshape(nb, 8, nt, 128).transpose(0, 2, 1, 3)

    nb_tc = nb // 2
    b_lo = nb_tc * 8
    parts = _build_sc(B, N, b_lo)(e5, t5, l4)

    ts = 4
    tc_partial = pl.pallas_call(
        _tc_body,
        grid=(nb_tc, nt // ts),
        in_specs=[
            pl.BlockSpec((5, 1, ts, 8, 128), lambda tr, tc: (0, tr, tc, 0, 0)),
            pl.BlockSpec((5, 1, ts, 8, 128), lambda tr, tc: (0, tr, tc, 0, 0)),
            pl.BlockSpec((1, ts, 8, 128), lambda tr, tc: (tr, tc, 0, 0)),
        ],
        out_specs=pl.BlockSpec(memory_space=pltpu.SMEM),
        out_shape=jax.ShapeDtypeStruct((1, 2), jnp.float32),
    )(e5, t5, l4)

    finish = pl.pallas_call(
        _finish_body,
        out_shape=jax.ShapeDtypeStruct((1, 1), jnp.float32),
        out_specs=pl.BlockSpec(memory_space=pltpu.SMEM),
    )(parts.reshape(8, _NW * 2 * _L // 8), tc_partial)
    return finish[0, 0]


# speedup vs baseline: 1.7248x; 1.7248x over previous
import functools

import jax
import jax.numpy as jnp
from jax import lax
from jax.experimental import pallas as pl
from jax.experimental.pallas import tpu as pltpu
from jax.experimental.pallas import tpu_sc as plsc

_NC = 2
_NS = 16
_NW = _NC * _NS
_L = 16
_TC_FRAC_NUM = 16
_TC_TS = 8


def _kld_terms(dxo, dyo, dlo, dso, ao, dxt, dyt, dlt, dst, at_):
    aa = ao * ao + 1.0
    bb = at_ * at_ + 1.0
    cc = ao * at_ + 1.0
    ss = ao - at_
    elt = jnp.exp(dlt + dlt)
    est = jnp.exp(dst + dst)
    ielo = jnp.exp(-(dlo + dlo))
    ieso = jnp.exp(-(dso + dso))
    t12 = elt * ielo + est * ieso
    t34 = elt * ieso + est * ielo
    tn = cc * cc * t12 + ss * ss * t34
    dx = dxo - dxt
    dy = dyo - dyt
    u = dx + ao * dy
    v = dy - ao * dx
    dn = u * u * ielo + v * v * ieso
    qh = 0.5 / (aa * bb)
    det = (dlo - dlt) + (dso - dst)
    return (tn + (4.0 * bb) * dn) * qh + det


@functools.lru_cache(maxsize=None)
def _build_sc(B, N, t_lo):
    nt = N // 128
    npt = (nt - t_lo) // _NW
    mesh = plsc.VectorSubcoreMesh(core_axis_name="c", subcore_axis_name="s")

    @functools.partial(
        pl.kernel,
        out_type=jax.ShapeDtypeStruct((_NW * 2 * _L,), jnp.float32),
        mesh=mesh,
        compiler_params=pltpu.CompilerParams(needs_layout_passes=False),
        scratch_types=[
            pltpu.VMEM((5 * npt, 128), jnp.float32),
            pltpu.VMEM((5 * npt, 128), jnp.float32),
            pltpu.VMEM((5 * npt, 128), jnp.float32),
            pltpu.VMEM((5 * npt, 128), jnp.float32),
            pltpu.VMEM((npt, 128), jnp.int32),
            pltpu.VMEM((npt, 128), jnp.int32),
            pltpu.VMEM((2 * _L,), jnp.float32),
            pltpu.SemaphoreType.DMA,
            pltpu.SemaphoreType.DMA,
        ],
    )
    def sc_kern(e_hbm, t_hbm, l_hbm, part_hbm,
                ebuf0, ebuf1, tbuf0, tbuf1, lbuf0, lbuf1,
                pbuf, sem0, sem1):
        wid = lax.axis_index("s") * _NC + lax.axis_index("c")
        tc0 = t_lo + wid * npt

        def issue(b, eb, tb, lb, sem):
            tr = b >> 3
            sl = b & 7
            pltpu.async_copy(
                e_hbm.at[:, tr, pl.ds(tc0, npt), sl, :], eb.reshape(5, npt, 128), sem)
            pltpu.async_copy(
                t_hbm.at[:, tr, pl.ds(tc0, npt), sl, :], tb.reshape(5, npt, 128), sem)
            pltpu.async_copy(l_hbm.at[tr, pl.ds(tc0, npt), sl, :], lb, sem)

        def drain(eb, tb, lb, sem):
            pltpu.make_async_copy(
                e_hbm.at[:, 0, pl.ds(0, npt), 0, :], eb.reshape(5, npt, 128), sem).wait()
            pltpu.make_async_copy(
                t_hbm.at[:, 0, pl.ds(0, npt), 0, :], tb.reshape(5, npt, 128), sem).wait()
            pltpu.make_async_copy(l_hbm.at[0, pl.ds(0, npt), 0, :], lb, sem).wait()

        def compute(eb, tb, lb, acc):
            def tile_body(k, acc2):
                ak, ac = acc2
                i = k >> 1
                jb = (k & 1) * (4 * _L)
                for j4 in range(4):
                    j = jb + j4 * _L
                    kld = _kld_terms(
                        eb[i, pl.ds(j, _L)],
                        eb[i + npt, pl.ds(j, _L)],
                        eb[i + 2 * npt, pl.ds(j, _L)],
                        eb[i + 3 * npt, pl.ds(j, _L)],
                        eb[i + 4 * npt, pl.ds(j, _L)],
                        tb[i, pl.ds(j, _L)],
                        tb[i + npt, pl.ds(j, _L)],
                        tb[i + 2 * npt, pl.ds(j, _L)],
                        tb[i + 3 * npt, pl.ds(j, _L)],
                        tb[i + 4 * npt, pl.ds(j, _L)],
                    )
                    lf = lb[i, pl.ds(j, _L)].astype(jnp.float32)
                    ak = ak + kld * lf
                    ac = ac + lf
                return (ak, ac)

            return lax.fori_loop(0, npt * 2, tile_body, acc)

        zero = jnp.zeros((_L,), jnp.float32)
        issue(0, ebuf0, tbuf0, lbuf0, sem0)

        def g_body(g, acc):
            b0 = g * 2
            issue(b0 + 1, ebuf1, tbuf1, lbuf1, sem1)
            drain(ebuf0, tbuf0, lbuf0, sem0)
            acc = compute(ebuf0, tbuf0, lbuf0, acc)

            @pl.when(b0 + 2 < B)
            def _():
                issue(b0 + 2, ebuf0, tbuf0, lbuf0, sem0)

            drain(ebuf1, tbuf1, lbuf1, sem1)
            return compute(ebuf1, tbuf1, lbuf1, acc)

        acc_k, acc_c = lax.fori_loop(0, B // 2, g_body, (zero, zero))
        pbuf[pl.ds(0, _L)] = acc_k
        pbuf[pl.ds(_L, _L)] = acc_c
        pltpu.sync_copy(pbuf, part_hbm.at[pl.ds(wid * 2 * _L, 2 * _L)])

    return sc_kern


def _tc_body(ngrid, e_ref, t_ref, l_ref, o_ref, acc_ref):
    tr = pl.program_id(0)
    tc = pl.program_id(1)

    @pl.when((tr == 0) & (tc == 0))
    def _():
        acc_ref[...] = jnp.zeros_like(acc_ref)

    kld = _kld_terms(
        e_ref[0, 0], e_ref[1, 0], e_ref[2, 0], e_ref[3, 0], e_ref[4, 0],
        t_ref[0, 0], t_ref[1, 0], t_ref[2, 0], t_ref[3, 0], t_ref[4, 0],
    )
    lf = l_ref[0].astype(jnp.float32)
    acc_ref[0] += jnp.sum(kld * lf, axis=0)
    acc_ref[1] += jnp.sum(lf, axis=0)

    @pl.when((tr == ngrid[0] - 1) & (tc == ngrid[1] - 1))
    def _():
        o_ref[0, 0] = jnp.sum(acc_ref[0])
        o_ref[0, 1] = jnp.sum(acc_ref[1])


def _finish_body(p_ref, tc_ref, o_ref):
    x = p_ref[...]
    lane = lax.broadcasted_iota(jnp.int32, x.shape, 1)
    is_k = (lane % (2 * _L)) < _L
    sk = jnp.sum(jnp.where(is_k, x, 0.0)) + tc_ref[0, 0]
    sc = jnp.sum(jnp.where(is_k, 0.0, x)) + tc_ref[0, 1]
    o_ref[0, 0] = sk / sc - 1.0


def kernel(out_ellipse, labels, ellipse_targets, anchors):
    B, N, F = out_ellipse.shape
    nt = N // 128
    nb = B // 8
    e5 = out_ellipse.transpose(2, 0, 1).reshape(F, nb, 8, nt, 128).transpose(0, 1, 3, 2, 4)
    t5 = ellipse_targets.transpose(2, 0, 1).reshape(F, nb, 8, nt, 128).transpose(0, 1, 3, 2, 4)
    l4 = labels.reshape(nb, 8, nt, 128).transpose(0, 2, 1, 3)

    t_lo = nt * _TC_FRAC_NUM // 32
    parts = _build_sc(B, N, t_lo)(e5, t5, l4)

    ntc = t_lo // _TC_TS
    tc_partial = pl.pallas_call(
        functools.partial(_tc_body, (nb, ntc)),
        grid=(nb, ntc),
        in_specs=[
            pl.BlockSpec((5, 1, _TC_TS, 8, 128), lambda tr, tc: (0, tr, tc, 0, 0)),
            pl.BlockSpec((5, 1, _TC_TS, 8, 128), lambda tr, tc: (0, tr, tc, 0, 0)),
            pl.BlockSpec((1, _TC_TS, 8, 128), lambda tr, tc: (tr, tc, 0, 0)),
        ],
        out_specs=pl.BlockSpec(memory_space=pltpu.SMEM),
        out_shape=jax.ShapeDtypeStruct((1, 2), jnp.float32),
        scratch_shapes=[pltpu.VMEM((2, 8, 128), jnp.float32)],
    )(e5, t5, l4)

    finish = pl.pallas_call(
        _finish_body,
        out_shape=jax.ShapeDtypeStruct((1, 1), jnp.float32),
        out_specs=pl.BlockSpec(memory_space=pltpu.SMEM),
    )(parts.reshape(8, _NW * 2 * _L // 8), tc_partial)
    return finish[0, 0]


# speedup vs baseline: 1.7254x; 1.0004x over previous
import functools

import jax
import jax.numpy as jnp
from jax import lax
from jax.experimental import pallas as pl
from jax.experimental.pallas import tpu as pltpu
from jax.experimental.pallas import tpu_sc as plsc

_NC = 2
_NS = 16
_NW = _NC * _NS
_L = 16
_TC_FRAC_NUM = 16
_TC_TS = 8


def _kld_terms(dxo, dyo, dlo, dso, ao, dxt, dyt, dlt, dst, at_):
    aa = ao * ao + 1.0
    bb = at_ * at_ + 1.0
    cc = ao * at_ + 1.0
    ss = ao - at_
    elt = jnp.exp(dlt + dlt)
    est = jnp.exp(dst + dst)
    ielo = jnp.exp(-(dlo + dlo))
    ieso = jnp.exp(-(dso + dso))
    t12 = elt * ielo + est * ieso
    t34 = elt * ieso + est * ielo
    tn = cc * cc * t12 + ss * ss * t34
    dx = dxo - dxt
    dy = dyo - dyt
    u = dx + ao * dy
    v = dy - ao * dx
    dn = u * u * ielo + v * v * ieso
    qh = 0.5 / (aa * bb)
    det = (dlo - dlt) + (dso - dst)
    return (tn + (4.0 * bb) * dn) * qh + det


@functools.lru_cache(maxsize=None)
def _build_sc(B, N, t_lo):
    nt = N // 128
    npt = (nt - t_lo) // _NW
    mesh = plsc.VectorSubcoreMesh(core_axis_name="c", subcore_axis_name="s")

    @functools.partial(
        pl.kernel,
        out_type=jax.ShapeDtypeStruct((_NW * 2 * _L,), jnp.float32),
        mesh=mesh,
        compiler_params=pltpu.CompilerParams(needs_layout_passes=False),
        scratch_types=[
            pltpu.VMEM((5 * npt, 128), jnp.float32),
            pltpu.VMEM((5 * npt, 128), jnp.float32),
            pltpu.VMEM((5 * npt, 128), jnp.float32),
            pltpu.VMEM((5 * npt, 128), jnp.float32),
            pltpu.VMEM((npt, 128), jnp.int32),
            pltpu.VMEM((npt, 128), jnp.int32),
            pltpu.VMEM((2 * _L,), jnp.float32),
            pltpu.SemaphoreType.DMA,
            pltpu.SemaphoreType.DMA,
        ],
    )
    def sc_kern(e_hbm, t_hbm, l_hbm, part_hbm,
                ebuf0, ebuf1, tbuf0, tbuf1, lbuf0, lbuf1,
                pbuf, sem0, sem1):
        wid = lax.axis_index("s") * _NC + lax.axis_index("c")
        tc0 = t_lo + wid * npt

        def issue(b, eb, tb, lb, sem):
            tr = b >> 3
            sl = b & 7
            pltpu.async_copy(
                e_hbm.at[:, tr, pl.ds(tc0, npt), sl, :], eb.reshape(5, npt, 128), sem)
            pltpu.async_copy(
                t_hbm.at[:, tr, pl.ds(tc0, npt), sl, :], tb.reshape(5, npt, 128), sem)
            pltpu.async_copy(l_hbm.at[tr, pl.ds(tc0, npt), sl, :], lb, sem)

        def drain(eb, tb, lb, sem):
            pltpu.make_async_copy(
                e_hbm.at[:, 0, pl.ds(0, npt), 0, :], eb.reshape(5, npt, 128), sem).wait()
            pltpu.make_async_copy(
                t_hbm.at[:, 0, pl.ds(0, npt), 0, :], tb.reshape(5, npt, 128), sem).wait()
            pltpu.make_async_copy(l_hbm.at[0, pl.ds(0, npt), 0, :], lb, sem).wait()

        def compute(eb, tb, lb, acc):
            def tile_body(k, acc2):
                ak, ac = acc2
                i = k >> 1
                jb = (k & 1) * (4 * _L)
                for j4 in range(4):
                    j = jb + j4 * _L
                    kld = _kld_terms(
                        eb[i, pl.ds(j, _L)],
                        eb[i + npt, pl.ds(j, _L)],
                        eb[i + 2 * npt, pl.ds(j, _L)],
                        eb[i + 3 * npt, pl.ds(j, _L)],
                        eb[i + 4 * npt, pl.ds(j, _L)],
                        tb[i, pl.ds(j, _L)],
                        tb[i + npt, pl.ds(j, _L)],
                        tb[i + 2 * npt, pl.ds(j, _L)],
                        tb[i + 3 * npt, pl.ds(j, _L)],
                        tb[i + 4 * npt, pl.ds(j, _L)],
                    )
                    lf = lb[i, pl.ds(j, _L)].astype(jnp.float32)
                    ak = ak + kld * lf
                    ac = ac + lf
                return (ak, ac)

            return lax.fori_loop(0, npt * 2, tile_body, acc)

        zero = jnp.zeros((_L,), jnp.float32)
        issue(0, ebuf0, tbuf0, lbuf0, sem0)

        def g_body(g, acc):
            b0 = g * 2
            issue(b0 + 1, ebuf1, tbuf1, lbuf1, sem1)
            drain(ebuf0, tbuf0, lbuf0, sem0)
            acc = compute(ebuf0, tbuf0, lbuf0, acc)

            @pl.when(b0 + 2 < B)
            def _():
                issue(b0 + 2, ebuf0, tbuf0, lbuf0, sem0)

            drain(ebuf1, tbuf1, lbuf1, sem1)
            return compute(ebuf1, tbuf1, lbuf1, acc)

        acc_k, acc_c = lax.fori_loop(0, B // 2, g_body, (zero, zero))
        pbuf[pl.ds(0, _L)] = acc_k
        pbuf[pl.ds(_L, _L)] = acc_c
        pltpu.sync_copy(pbuf, part_hbm.at[pl.ds(wid * 2 * _L, 2 * _L)])

    return sc_kern


def _tc_body(ngrid, e_ref, t_ref, l_ref, o_ref, acc_ref):
    tr = pl.program_id(0)
    tc = pl.program_id(1)

    @pl.when((tr == 0) & (tc == 0))
    def _():
        acc_ref[...] = jnp.zeros_like(acc_ref)

    ak = acc_ref[0]
    ac = acc_ref[1]
    for st in range(_TC_TS):
        kld = _kld_terms(
            e_ref[0, 0, st], e_ref[1, 0, st], e_ref[2, 0, st],
            e_ref[3, 0, st], e_ref[4, 0, st],
            t_ref[0, 0, st], t_ref[1, 0, st], t_ref[2, 0, st],
            t_ref[3, 0, st], t_ref[4, 0, st],
        )
        lf = l_ref[0, st].astype(jnp.float32)
        ak = ak + kld * lf
        ac = ac + lf
    acc_ref[0] = ak
    acc_ref[1] = ac

    @pl.when((tr == ngrid[0] - 1) & (tc == ngrid[1] - 1))
    def _():
        o_ref[0, 0] = jnp.sum(acc_ref[0])
        o_ref[0, 1] = jnp.sum(acc_ref[1])


def _finish_body(p_ref, tc_ref, o_ref):
    x = p_ref[...]
    lane = lax.broadcasted_iota(jnp.int32, x.shape, 1)
    is_k = (lane % (2 * _L)) < _L
    sk = jnp.sum(jnp.where(is_k, x, 0.0)) + tc_ref[0, 0]
    sc = jnp.sum(jnp.where(is_k, 0.0, x)) + tc_ref[0, 1]
    o_ref[0, 0] = sk / sc - 1.0


def kernel(out_ellipse, labels, ellipse_targets, anchors):
    B, N, F = out_ellipse.shape
    nt = N // 128
    nb = B // 8
    e5 = out_ellipse.transpose(2, 0, 1).reshape(F, nb, 8, nt, 128).transpose(0, 1, 3, 2, 4)
    t5 = ellipse_targets.transpose(2, 0, 1).reshape(F, nb, 8, nt, 128).transpose(0, 1, 3, 2, 4)
    l4 = labels.reshape(nb, 8, nt, 128).transpose(0, 2, 1, 3)

    t_lo = nt * _TC_FRAC_NUM // 32
    parts = _build_sc(B, N, t_lo)(e5, t5, l4)

    ntc = t_lo // _TC_TS
    tc_partial = pl.pallas_call(
        functools.partial(_tc_body, (nb, ntc)),
        grid=(nb, ntc),
        in_specs=[
            pl.BlockSpec((5, 1, _TC_TS, 8, 128), lambda tr, tc: (0, tr, tc, 0, 0)),
            pl.BlockSpec((5, 1, _TC_TS, 8, 128), lambda tr, tc: (0, tr, tc, 0, 0)),
            pl.BlockSpec((1, _TC_TS, 8, 128), lambda tr, tc: (tr, tc, 0, 0)),
        ],
        out_specs=pl.BlockSpec(memory_space=pltpu.SMEM),
        out_shape=jax.ShapeDtypeStruct((1, 2), jnp.float32),
        scratch_shapes=[pltpu.VMEM((2, 8, 128), jnp.float32)],
    )(e5, t5, l4)

    finish = pl.pallas_call(
        _finish_body,
        out_shape=jax.ShapeDtypeStruct((1, 1), jnp.float32),
        out_specs=pl.BlockSpec(memory_space=pltpu.SMEM),
    )(parts.reshape(8, _NW * 2 * _L // 8), tc_partial)
    return finish[0, 0]


# speedup vs baseline: 2.4878x; 1.4418x over previous
import functools

import jax
import jax.numpy as jnp
from jax import lax
from jax.experimental import pallas as pl
from jax.experimental.pallas import tpu as pltpu
from jax.experimental.pallas import tpu_sc as plsc

_NC = 2
_NS = 16
_NW = _NC * _NS
_L = 16
_TC_FRAC_NUM = 16
_TC_TS = 96


def _kld_terms(dxo, dyo, dlo, dso, ao, dxt, dyt, dlt, dst, at_):
    aa = ao * ao + 1.0
    bb = at_ * at_ + 1.0
    cc = ao * at_ + 1.0
    ss = ao - at_
    elt = jnp.exp(dlt + dlt)
    est = jnp.exp(dst + dst)
    ielo = jnp.exp(-(dlo + dlo))
    ieso = jnp.exp(-(dso + dso))
    t12 = elt * ielo + est * ieso
    t34 = elt * ieso + est * ielo
    tn = cc * cc * t12 + ss * ss * t34
    dx = dxo - dxt
    dy = dyo - dyt
    u = dx + ao * dy
    v = dy - ao * dx
    dn = u * u * ielo + v * v * ieso
    qh = 0.5 / (aa * bb)
    det = (dlo - dlt) + (dso - dst)
    return (tn + (4.0 * bb) * dn) * qh + det


@functools.lru_cache(maxsize=None)
def _build_sc(B, N, t_lo):
    nt = N // 128
    npt = (nt - t_lo) // _NW
    mesh = plsc.VectorSubcoreMesh(core_axis_name="c", subcore_axis_name="s")

    @functools.partial(
        pl.kernel,
        out_type=jax.ShapeDtypeStruct((_NW * 2 * _L,), jnp.float32),
        mesh=mesh,
        compiler_params=pltpu.CompilerParams(needs_layout_passes=False),
        scratch_types=[
            pltpu.VMEM((5 * npt, 128), jnp.float32),
            pltpu.VMEM((5 * npt, 128), jnp.float32),
            pltpu.VMEM((5 * npt, 128), jnp.float32),
            pltpu.VMEM((5 * npt, 128), jnp.float32),
            pltpu.VMEM((npt, 128), jnp.int32),
            pltpu.VMEM((npt, 128), jnp.int32),
            pltpu.VMEM((2 * _L,), jnp.float32),
            pltpu.SemaphoreType.DMA,
            pltpu.SemaphoreType.DMA,
        ],
    )
    def sc_kern(e_hbm, t_hbm, l_hbm, part_hbm,
                ebuf0, ebuf1, tbuf0, tbuf1, lbuf0, lbuf1,
                pbuf, sem0, sem1):
        wid = lax.axis_index("s") * _NC + lax.axis_index("c")
        tc0 = t_lo + wid * npt

        def issue(b, eb, tb, lb, sem):
            tr = b >> 3
            sl = b & 7
            pltpu.async_copy(
                e_hbm.at[:, tr, pl.ds(tc0, npt), sl, :], eb.reshape(5, npt, 128), sem)
            pltpu.async_copy(
                t_hbm.at[:, tr, pl.ds(tc0, npt), sl, :], tb.reshape(5, npt, 128), sem)
            pltpu.async_copy(l_hbm.at[tr, pl.ds(tc0, npt), sl, :], lb, sem)

        def drain(eb, tb, lb, sem):
            pltpu.make_async_copy(
                e_hbm.at[:, 0, pl.ds(0, npt), 0, :], eb.reshape(5, npt, 128), sem).wait()
            pltpu.make_async_copy(
                t_hbm.at[:, 0, pl.ds(0, npt), 0, :], tb.reshape(5, npt, 128), sem).wait()
            pltpu.make_async_copy(l_hbm.at[0, pl.ds(0, npt), 0, :], lb, sem).wait()

        def compute(eb, tb, lb, acc):
            def tile_body(k, acc2):
                ak, ac = acc2
                i = k >> 1
                jb = (k & 1) * (4 * _L)
                for j4 in range(4):
                    j = jb + j4 * _L
                    kld = _kld_terms(
                        eb[i, pl.ds(j, _L)],
                        eb[i + npt, pl.ds(j, _L)],
                        eb[i + 2 * npt, pl.ds(j, _L)],
                        eb[i + 3 * npt, pl.ds(j, _L)],
                        eb[i + 4 * npt, pl.ds(j, _L)],
                        tb[i, pl.ds(j, _L)],
                        tb[i + npt, pl.ds(j, _L)],
                        tb[i + 2 * npt, pl.ds(j, _L)],
                        tb[i + 3 * npt, pl.ds(j, _L)],
                        tb[i + 4 * npt, pl.ds(j, _L)],
                    )
                    lf = lb[i, pl.ds(j, _L)].astype(jnp.float32)
                    ak = ak + kld * lf
                    ac = ac + lf
                return (ak, ac)

            return lax.fori_loop(0, npt * 2, tile_body, acc)

        zero = jnp.zeros((_L,), jnp.float32)
        issue(0, ebuf0, tbuf0, lbuf0, sem0)

        def g_body(g, acc):
            b0 = g * 2
            issue(b0 + 1, ebuf1, tbuf1, lbuf1, sem1)
            drain(ebuf0, tbuf0, lbuf0, sem0)
            acc = compute(ebuf0, tbuf0, lbuf0, acc)

            @pl.when(b0 + 2 < B)
            def _():
                issue(b0 + 2, ebuf0, tbuf0, lbuf0, sem0)

            drain(ebuf1, tbuf1, lbuf1, sem1)
            return compute(ebuf1, tbuf1, lbuf1, acc)

        acc_k, acc_c = lax.fori_loop(0, B // 2, g_body, (zero, zero))
        pbuf[pl.ds(0, _L)] = acc_k
        pbuf[pl.ds(_L, _L)] = acc_c
        pltpu.sync_copy(pbuf, part_hbm.at[pl.ds(wid * 2 * _L, 2 * _L)])

    return sc_kern


def _tc_body(ngrid, e_ref, t_ref, l_ref, o_ref, acc_ref):
    tr = pl.program_id(0)
    tc = pl.program_id(1)

    @pl.when((tr == 0) & (tc == 0))
    def _():
        acc_ref[...] = jnp.zeros_like(acc_ref)

    ak = acc_ref[0]
    ac = acc_ref[1]
    for st in range(_TC_TS):
        kld = _kld_terms(
            e_ref[0, 0, st], e_ref[1, 0, st], e_ref[2, 0, st],
            e_ref[3, 0, st], e_ref[4, 0, st],
            t_ref[0, 0, st], t_ref[1, 0, st], t_ref[2, 0, st],
            t_ref[3, 0, st], t_ref[4, 0, st],
        )
        lf = l_ref[0, st].astype(jnp.float32)
        ak = ak + kld * lf
        ac = ac + lf
    acc_ref[0] = ak
    acc_ref[1] = ac

    @pl.when((tr == ngrid[0] - 1) & (tc == ngrid[1] - 1))
    def _():
        o_ref[0, 0] = jnp.sum(acc_ref[0])
        o_ref[0, 1] = jnp.sum(acc_ref[1])


def _finish_body(p_ref, tc_ref, o_ref):
    x = p_ref[...]
    lane = lax.broadcasted_iota(jnp.int32, x.shape, 1)
    is_k = (lane % (2 * _L)) < _L
    sk = jnp.sum(jnp.where(is_k, x, 0.0)) + tc_ref[0, 0]
    sc = jnp.sum(jnp.where(is_k, 0.0, x)) + tc_ref[0, 1]
    o_ref[0, 0] = sk / sc - 1.0


def kernel(out_ellipse, labels, ellipse_targets, anchors):
    B, N, F = out_ellipse.shape
    nt = N // 128
    nb = B // 8
    e5 = out_ellipse.transpose(2, 0, 1).reshape(F, nb, 8, nt, 128).transpose(0, 1, 3, 2, 4)
    t5 = ellipse_targets.transpose(2, 0, 1).reshape(F, nb, 8, nt, 128).transpose(0, 1, 3, 2, 4)
    l4 = labels.reshape(nb, 8, nt, 128).transpose(0, 2, 1, 3)

    t_lo = nt * _TC_FRAC_NUM // 32
    parts = _build_sc(B, N, t_lo)(e5, t5, l4)

    ntc = t_lo // _TC_TS
    tc_partial = pl.pallas_call(
        functools.partial(_tc_body, (nb, ntc)),
        grid=(nb, ntc),
        in_specs=[
            pl.BlockSpec((5, 1, _TC_TS, 8, 128), lambda tr, tc: (0, tr, tc, 0, 0)),
            pl.BlockSpec((5, 1, _TC_TS, 8, 128), lambda tr, tc: (0, tr, tc, 0, 0)),
            pl.BlockSpec((1, _TC_TS, 8, 128), lambda tr, tc: (tr, tc, 0, 0)),
        ],
        out_specs=pl.BlockSpec(memory_space=pltpu.SMEM),
        out_shape=jax.ShapeDtypeStruct((1, 2), jnp.float32),
        scratch_shapes=[pltpu.VMEM((2, 8, 128), jnp.float32)],
    )(e5, t5, l4)

    finish = pl.pallas_call(
        _finish_body,
        out_shape=jax.ShapeDtypeStruct((1, 1), jnp.float32),
        out_specs=pl.BlockSpec(memory_space=pltpu.SMEM),
    )(parts.reshape(8, _NW * 2 * _L // 8), tc_partial)
    return finish[0, 0]


# speedup vs baseline: 2.7773x; 1.1164x over previous
import functools

import jax
import jax.numpy as jnp
from jax import lax
from jax.experimental import pallas as pl
from jax.experimental.pallas import tpu as pltpu
from jax.experimental.pallas import tpu_sc as plsc

_NC = 2
_NS = 16
_NW = _NC * _NS
_L = 16
_TC_FRAC_NUM = 24
_TC_TS = 96


def _kld_terms(dxo, dyo, dlo, dso, ao, dxt, dyt, dlt, dst, at_):
    aa = ao * ao + 1.0
    bb = at_ * at_ + 1.0
    cc = ao * at_ + 1.0
    ss = ao - at_
    elt = jnp.exp(dlt + dlt)
    est = jnp.exp(dst + dst)
    ielo = jnp.exp(-(dlo + dlo))
    ieso = jnp.exp(-(dso + dso))
    t12 = elt * ielo + est * ieso
    t34 = elt * ieso + est * ielo
    tn = cc * cc * t12 + ss * ss * t34
    dx = dxo - dxt
    dy = dyo - dyt
    u = dx + ao * dy
    v = dy - ao * dx
    dn = u * u * ielo + v * v * ieso
    qh = 0.5 / (aa * bb)
    det = (dlo - dlt) + (dso - dst)
    return (tn + (4.0 * bb) * dn) * qh + det


@functools.lru_cache(maxsize=None)
def _build_sc(B, N, t_lo):
    nt = N // 128
    npt = (nt - t_lo) // _NW
    mesh = plsc.VectorSubcoreMesh(core_axis_name="c", subcore_axis_name="s")

    @functools.partial(
        pl.kernel,
        out_type=jax.ShapeDtypeStruct((_NW * 2 * _L,), jnp.float32),
        mesh=mesh,
        compiler_params=pltpu.CompilerParams(needs_layout_passes=False),
        scratch_types=[
            pltpu.VMEM((5 * npt, 128), jnp.float32),
            pltpu.VMEM((5 * npt, 128), jnp.float32),
            pltpu.VMEM((5 * npt, 128), jnp.float32),
            pltpu.VMEM((5 * npt, 128), jnp.float32),
            pltpu.VMEM((npt, 128), jnp.int32),
            pltpu.VMEM((npt, 128), jnp.int32),
            pltpu.VMEM((2 * _L,), jnp.float32),
            pltpu.SemaphoreType.DMA,
            pltpu.SemaphoreType.DMA,
        ],
    )
    def sc_kern(e_hbm, t_hbm, l_hbm, part_hbm,
                ebuf0, ebuf1, tbuf0, tbuf1, lbuf0, lbuf1,
                pbuf, sem0, sem1):
        wid = lax.axis_index("s") * _NC + lax.axis_index("c")
        tc0 = t_lo + wid * npt

        def issue(b, eb, tb, lb, sem):
            tr = b >> 3
            sl = b & 7
            pltpu.async_copy(
                e_hbm.at[:, tr, pl.ds(tc0, npt), sl, :], eb.reshape(5, npt, 128), sem)
            pltpu.async_copy(
                t_hbm.at[:, tr, pl.ds(tc0, npt), sl, :], tb.reshape(5, npt, 128), sem)
            pltpu.async_copy(l_hbm.at[tr, pl.ds(tc0, npt), sl, :], lb, sem)

        def drain(eb, tb, lb, sem):
            pltpu.make_async_copy(
                e_hbm.at[:, 0, pl.ds(0, npt), 0, :], eb.reshape(5, npt, 128), sem).wait()
            pltpu.make_async_copy(
                t_hbm.at[:, 0, pl.ds(0, npt), 0, :], tb.reshape(5, npt, 128), sem).wait()
            pltpu.make_async_copy(l_hbm.at[0, pl.ds(0, npt), 0, :], lb, sem).wait()

        def compute(eb, tb, lb, acc):
            def tile_body(k, acc2):
                ak, ac = acc2
                i = k >> 1
                jb = (k & 1) * (4 * _L)
                for j4 in range(4):
                    j = jb + j4 * _L
                    kld = _kld_terms(
                        eb[i, pl.ds(j, _L)],
                        eb[i + npt, pl.ds(j, _L)],
                        eb[i + 2 * npt, pl.ds(j, _L)],
                        eb[i + 3 * npt, pl.ds(j, _L)],
                        eb[i + 4 * npt, pl.ds(j, _L)],
                        tb[i, pl.ds(j, _L)],
                        tb[i + npt, pl.ds(j, _L)],
                        tb[i + 2 * npt, pl.ds(j, _L)],
                        tb[i + 3 * npt, pl.ds(j, _L)],
                        tb[i + 4 * npt, pl.ds(j, _L)],
                    )
                    lf = lb[i, pl.ds(j, _L)].astype(jnp.float32)
                    ak = ak + kld * lf
                    ac = ac + lf
                return (ak, ac)

            return lax.fori_loop(0, npt * 2, tile_body, acc)

        zero = jnp.zeros((_L,), jnp.float32)
        issue(0, ebuf0, tbuf0, lbuf0, sem0)

        def g_body(g, acc):
            b0 = g * 2
            issue(b0 + 1, ebuf1, tbuf1, lbuf1, sem1)
            drain(ebuf0, tbuf0, lbuf0, sem0)
            acc = compute(ebuf0, tbuf0, lbuf0, acc)

            @pl.when(b0 + 2 < B)
            def _():
                issue(b0 + 2, ebuf0, tbuf0, lbuf0, sem0)

            drain(ebuf1, tbuf1, lbuf1, sem1)
            return compute(ebuf1, tbuf1, lbuf1, acc)

        acc_k, acc_c = lax.fori_loop(0, B // 2, g_body, (zero, zero))
        pbuf[pl.ds(0, _L)] = acc_k
        pbuf[pl.ds(_L, _L)] = acc_c
        pltpu.sync_copy(pbuf, part_hbm.at[pl.ds(wid * 2 * _L, 2 * _L)])

    return sc_kern


def _tc_body(ngrid, e_ref, t_ref, l_ref, o_ref, acc_ref):
    tr = pl.program_id(0)
    tc = pl.program_id(1)

    @pl.when((tr == 0) & (tc == 0))
    def _():
        acc_ref[...] = jnp.zeros_like(acc_ref)

    ak = acc_ref[0]
    ac = acc_ref[1]
    for st in range(_TC_TS):
        kld = _kld_terms(
            e_ref[0, 0, st], e_ref[1, 0, st], e_ref[2, 0, st],
            e_ref[3, 0, st], e_ref[4, 0, st],
            t_ref[0, 0, st], t_ref[1, 0, st], t_ref[2, 0, st],
            t_ref[3, 0, st], t_ref[4, 0, st],
        )
        lf = l_ref[0, st].astype(jnp.float32)
        ak = ak + kld * lf
        ac = ac + lf
    acc_ref[0] = ak
    acc_ref[1] = ac

    @pl.when((tr == ngrid[0] - 1) & (tc == ngrid[1] - 1))
    def _():
        o_ref[0, 0] = jnp.sum(acc_ref[0])
        o_ref[0, 1] = jnp.sum(acc_ref[1])


def _finish_body(p_ref, tc_ref, o_ref):
    x = p_ref[...]
    lane = lax.broadcasted_iota(jnp.int32, x.shape, 1)
    is_k = (lane % (2 * _L)) < _L
    sk = jnp.sum(jnp.where(is_k, x, 0.0)) + tc_ref[0, 0]
    sc = jnp.sum(jnp.where(is_k, 0.0, x)) + tc_ref[0, 1]
    o_ref[0, 0] = sk / sc - 1.0


def kernel(out_ellipse, labels, ellipse_targets, anchors):
    B, N, F = out_ellipse.shape
    nt = N // 128
    nb = B // 8
    e5 = out_ellipse.transpose(2, 0, 1).reshape(F, nb, 8, nt, 128).transpose(0, 1, 3, 2, 4)
    t5 = ellipse_targets.transpose(2, 0, 1).reshape(F, nb, 8, nt, 128).transpose(0, 1, 3, 2, 4)
    l4 = labels.reshape(nb, 8, nt, 128).transpose(0, 2, 1, 3)

    t_lo = nt * _TC_FRAC_NUM // 32
    parts = _build_sc(B, N, t_lo)(e5, t5, l4)

    ntc = t_lo // _TC_TS
    tc_partial = pl.pallas_call(
        functools.partial(_tc_body, (nb, ntc)),
        grid=(nb, ntc),
        in_specs=[
            pl.BlockSpec((5, 1, _TC_TS, 8, 128), lambda tr, tc: (0, tr, tc, 0, 0)),
            pl.BlockSpec((5, 1, _TC_TS, 8, 128), lambda tr, tc: (0, tr, tc, 0, 0)),
            pl.BlockSpec((1, _TC_TS, 8, 128), lambda tr, tc: (tr, tc, 0, 0)),
        ],
        out_specs=pl.BlockSpec(memory_space=pltpu.SMEM),
        out_shape=jax.ShapeDtypeStruct((1, 2), jnp.float32),
        scratch_shapes=[pltpu.VMEM((2, 8, 128), jnp.float32)],
    )(e5, t5, l4)

    finish = pl.pallas_call(
        _finish_body,
        out_shape=jax.ShapeDtypeStruct((1, 1), jnp.float32),
        out_specs=pl.BlockSpec(memory_space=pltpu.SMEM),
    )(parts.reshape(8, _NW * 2 * _L // 8), tc_partial)
    return finish[0, 0]


# speedup vs baseline: 2.8125x; 1.0127x over previous
import functools

import jax
import jax.numpy as jnp
from jax import lax
from jax.experimental import pallas as pl
from jax.experimental.pallas import tpu as pltpu
from jax.experimental.pallas import tpu_sc as plsc

_NC = 2
_NS = 16
_NW = _NC * _NS
_L = 16
_TC_FRAC_NUM = 24
_TC_TS = 144


def _kld_terms(dxo, dyo, dlo, dso, ao, dxt, dyt, dlt, dst, at_):
    aa = ao * ao + 1.0
    bb = at_ * at_ + 1.0
    cc = ao * at_ + 1.0
    ss = ao - at_
    elt = jnp.exp(dlt + dlt)
    est = jnp.exp(dst + dst)
    ielo = jnp.exp(-(dlo + dlo))
    ieso = jnp.exp(-(dso + dso))
    t12 = elt * ielo + est * ieso
    t34 = elt * ieso + est * ielo
    tn = cc * cc * t12 + ss * ss * t34
    dx = dxo - dxt
    dy = dyo - dyt
    u = dx + ao * dy
    v = dy - ao * dx
    dn = u * u * ielo + v * v * ieso
    qh = 0.5 / (aa * bb)
    det = (dlo - dlt) + (dso - dst)
    return (tn + (4.0 * bb) * dn) * qh + det


@functools.lru_cache(maxsize=None)
def _build_sc(B, N, t_lo):
    nt = N // 128
    npt = (nt - t_lo) // _NW
    mesh = plsc.VectorSubcoreMesh(core_axis_name="c", subcore_axis_name="s")

    @functools.partial(
        pl.kernel,
        out_type=jax.ShapeDtypeStruct((_NW * 2 * _L,), jnp.float32),
        mesh=mesh,
        compiler_params=pltpu.CompilerParams(needs_layout_passes=False),
        scratch_types=[
            pltpu.VMEM((5 * npt, 128), jnp.float32),
            pltpu.VMEM((5 * npt, 128), jnp.float32),
            pltpu.VMEM((5 * npt, 128), jnp.float32),
            pltpu.VMEM((5 * npt, 128), jnp.float32),
            pltpu.VMEM((npt, 128), jnp.int32),
            pltpu.VMEM((npt, 128), jnp.int32),
            pltpu.VMEM((2 * _L,), jnp.float32),
            pltpu.SemaphoreType.DMA,
            pltpu.SemaphoreType.DMA,
        ],
    )
    def sc_kern(e_hbm, t_hbm, l_hbm, part_hbm,
                ebuf0, ebuf1, tbuf0, tbuf1, lbuf0, lbuf1,
                pbuf, sem0, sem1):
        wid = lax.axis_index("s") * _NC + lax.axis_index("c")
        tc0 = t_lo + wid * npt

        def issue(b, eb, tb, lb, sem):
            tr = b >> 3
            sl = b & 7
            pltpu.async_copy(
                e_hbm.at[:, tr, pl.ds(tc0, npt), sl, :], eb.reshape(5, npt, 128), sem)
            pltpu.async_copy(
                t_hbm.at[:, tr, pl.ds(tc0, npt), sl, :], tb.reshape(5, npt, 128), sem)
            pltpu.async_copy(l_hbm.at[tr, pl.ds(tc0, npt), sl, :], lb, sem)

        def drain(eb, tb, lb, sem):
            pltpu.make_async_copy(
                e_hbm.at[:, 0, pl.ds(0, npt), 0, :], eb.reshape(5, npt, 128), sem).wait()
            pltpu.make_async_copy(
                t_hbm.at[:, 0, pl.ds(0, npt), 0, :], tb.reshape(5, npt, 128), sem).wait()
            pltpu.make_async_copy(l_hbm.at[0, pl.ds(0, npt), 0, :], lb, sem).wait()

        def compute(eb, tb, lb, acc):
            def tile_body(k, acc2):
                ak, ac = acc2
                i = k >> 1
                jb = (k & 1) * (4 * _L)
                for j4 in range(4):
                    j = jb + j4 * _L
                    kld = _kld_terms(
                        eb[i, pl.ds(j, _L)],
                        eb[i + npt, pl.ds(j, _L)],
                        eb[i + 2 * npt, pl.ds(j, _L)],
                        eb[i + 3 * npt, pl.ds(j, _L)],
                        eb[i + 4 * npt, pl.ds(j, _L)],
                        tb[i, pl.ds(j, _L)],
                        tb[i + npt, pl.ds(j, _L)],
                        tb[i + 2 * npt, pl.ds(j, _L)],
                        tb[i + 3 * npt, pl.ds(j, _L)],
                        tb[i + 4 * npt, pl.ds(j, _L)],
                    )
                    lf = lb[i, pl.ds(j, _L)].astype(jnp.float32)
                    ak = ak + kld * lf
                    ac = ac + lf
                return (ak, ac)

            return lax.fori_loop(0, npt * 2, tile_body, acc)

        zero = jnp.zeros((_L,), jnp.float32)
        issue(0, ebuf0, tbuf0, lbuf0, sem0)

        def g_body(g, acc):
            b0 = g * 2
            issue(b0 + 1, ebuf1, tbuf1, lbuf1, sem1)
            drain(ebuf0, tbuf0, lbuf0, sem0)
            acc = compute(ebuf0, tbuf0, lbuf0, acc)

            @pl.when(b0 + 2 < B)
            def _():
                issue(b0 + 2, ebuf0, tbuf0, lbuf0, sem0)

            drain(ebuf1, tbuf1, lbuf1, sem1)
            return compute(ebuf1, tbuf1, lbuf1, acc)

        acc_k, acc_c = lax.fori_loop(0, B // 2, g_body, (zero, zero))
        pbuf[pl.ds(0, _L)] = acc_k
        pbuf[pl.ds(_L, _L)] = acc_c
        pltpu.sync_copy(pbuf, part_hbm.at[pl.ds(wid * 2 * _L, 2 * _L)])

    return sc_kern


def _tc_body(ngrid, e_ref, t_ref, l_ref, o_ref, acc_ref):
    tr = pl.program_id(0)
    tc = pl.program_id(1)

    @pl.when((tr == 0) & (tc == 0))
    def _():
        acc_ref[...] = jnp.zeros_like(acc_ref)

    ak = acc_ref[0]
    ac = acc_ref[1]
    for st in range(_TC_TS):
        kld = _kld_terms(
            e_ref[0, 0, st], e_ref[1, 0, st], e_ref[2, 0, st],
            e_ref[3, 0, st], e_ref[4, 0, st],
            t_ref[0, 0, st], t_ref[1, 0, st], t_ref[2, 0, st],
            t_ref[3, 0, st], t_ref[4, 0, st],
        )
        lf = l_ref[0, st].astype(jnp.float32)
        ak = ak + kld * lf
        ac = ac + lf
    acc_ref[0] = ak
    acc_ref[1] = ac

    @pl.when((tr == ngrid[0] - 1) & (tc == ngrid[1] - 1))
    def _():
        o_ref[0, 0] = jnp.sum(acc_ref[0])
        o_ref[0, 1] = jnp.sum(acc_ref[1])


def _finish_body(p_ref, tc_ref, o_ref):
    x = p_ref[...]
    lane = lax.broadcasted_iota(jnp.int32, x.shape, 1)
    is_k = (lane % (2 * _L)) < _L
    sk = jnp.sum(jnp.where(is_k, x, 0.0)) + tc_ref[0, 0]
    sc = jnp.sum(jnp.where(is_k, 0.0, x)) + tc_ref[0, 1]
    o_ref[0, 0] = sk / sc - 1.0


def kernel(out_ellipse, labels, ellipse_targets, anchors):
    B, N, F = out_ellipse.shape
    nt = N // 128
    nb = B // 8
    e5 = out_ellipse.transpose(2, 0, 1).reshape(F, nb, 8, nt, 128).transpose(0, 1, 3, 2, 4)
    t5 = ellipse_targets.transpose(2, 0, 1).reshape(F, nb, 8, nt, 128).transpose(0, 1, 3, 2, 4)
    l4 = labels.reshape(nb, 8, nt, 128).transpose(0, 2, 1, 3)

    t_lo = nt * _TC_FRAC_NUM // 32
    parts = _build_sc(B, N, t_lo)(e5, t5, l4)

    ntc = t_lo // _TC_TS
    tc_partial = pl.pallas_call(
        functools.partial(_tc_body, (nb, ntc)),
        grid=(nb, ntc),
        in_specs=[
            pl.BlockSpec((5, 1, _TC_TS, 8, 128), lambda tr, tc: (0, tr, tc, 0, 0)),
            pl.BlockSpec((5, 1, _TC_TS, 8, 128), lambda tr, tc: (0, tr, tc, 0, 0)),
            pl.BlockSpec((1, _TC_TS, 8, 128), lambda tr, tc: (tr, tc, 0, 0)),
        ],
        out_specs=pl.BlockSpec(memory_space=pltpu.SMEM),
        out_shape=jax.ShapeDtypeStruct((1, 2), jnp.float32),
        scratch_shapes=[pltpu.VMEM((2, 8, 128), jnp.float32)],
    )(e5, t5, l4)

    finish = pl.pallas_call(
        _finish_body,
        out_shape=jax.ShapeDtypeStruct((1, 1), jnp.float32),
        out_specs=pl.BlockSpec(memory_space=pltpu.SMEM),
    )(parts.reshape(8, _NW * 2 * _L // 8), tc_partial)
    return finish[0, 0]
